# baseline (speedup 1.0000x reference)
import jax
import jax.numpy as jnp
from jax import lax
from jax.experimental import pallas as pl
from jax.experimental.pallas import tpu as pltpu
from jax.experimental.pallas import tpu_sc as plsc

N_BITS = 20
BATCH = 16384
TABLE_SIZE = 1 << N_BITS

TC_ROWS = 16
SC_ROWS = N_BITS - TC_ROWS

NC = 2
NS = 16
NW = NC * NS

PW = 131072
PIECES = TABLE_SIZE // PW

CW = 2048
NCK = TABLE_SIZE // CW
CK_W = NCK // NW

TOK_W = BATCH // NW
CHUNK = 128
GROUP = 16
CPR = TOK_W // CHUNK


def _tc_relayout_body(table_ref, staging_ref):
    def body(sem):
        copies = []
        for p in range(TC_ROWS):
            for c in range(PIECES):
                copies.append(pltpu.make_async_copy(
                    table_ref.at[p, pl.ds(c * PW, PW)],
                    staging_ref.at[pl.ds(p * TABLE_SIZE + c * PW, PW)],
                    sem))
        for cp in copies:
            cp.start()
        for cp in copies:
            cp.wait()

    pl.run_scoped(body, pltpu.SemaphoreType.DMA)


def _sc_relayout_body(table_ref, staging_ref, buf0, buf1, rsem, wsem):
    wid = lax.axis_index("s") * NC + lax.axis_index("c")
    ck0 = wid * CK_W
    bufs = (buf0, buf1)

    def read(k, buf):
        col = pl.multiple_of((ck0 + lax.rem(k, CK_W)) * CW, CW)
        return pltpu.async_copy(
            table_ref.at[pl.ds(TC_ROWS, SC_ROWS), pl.ds(col, CW)],
            buf, rsem)

    def read_wait(k, buf):
        col = pl.multiple_of((ck0 + lax.rem(k, CK_W)) * CW, CW)
        pltpu.make_async_copy(
            table_ref.at[pl.ds(TC_ROWS, SC_ROWS), pl.ds(col, CW)],
            buf, rsem).wait()

    read(0, buf0)
    read(1, buf1)

    def step(k2, carry):
        for b in range(2):
            k = k2 * 2 + b
            cur = bufs[b]
            col = pl.multiple_of((ck0 + k) * CW, CW)
            read_wait(k, cur)
            writes = []
            for j in range(SC_ROWS):
                writes.append(pltpu.async_copy(
                    cur.at[j],
                    staging_ref.at[pl.ds(j * TABLE_SIZE + col, CW)],
                    wsem))
            for w in writes:
                w.wait()
            read(k + 2, cur)
        return carry

    lax.fori_loop(0, CK_W // 2, step, 0)
    read_wait(0, buf0)
    read_wait(1, buf1)


def _gather_body(stag_tc_ref, stag_sc_ref, bitst_ref, out_ref,
                 bits_v, idx_v, vals_v, sem):
    wid = lax.axis_index("s") * NC + lax.axis_index("c")
    tok0 = wid * TOK_W
    pltpu.sync_copy(bitst_ref.at[:, pl.ds(tok0, TOK_W)], bits_v)

    def compute_group(g, carry):
        o = pl.multiple_of(g * 16, 16)
        addr = jnp.zeros((16,), jnp.int32)
        for i in range(N_BITS):
            addr = addr + bits_v[i, pl.ds(o, 16)] * (1 << i)
        for p in range(N_BITS):
            rel = p if p < TC_ROWS else p - TC_ROWS
            idx_v[p, pl.ds(o, 16)] = addr + (rel << N_BITS)
        return carry

    lax.fori_loop(0, TOK_W // 16, compute_group, 0)

    def gather_group_tc(t, carry):
        copies = []
        for u in range(GROUP):
            k = t * GROUP + u
            p = lax.div(k, CPR)
            o = pl.multiple_of(lax.rem(k, CPR) * CHUNK, CHUNK)
            copies.append(pltpu.async_copy(
                stag_tc_ref.at[idx_v.at[p, pl.ds(o, CHUNK)]],
                vals_v.at[p, pl.ds(o, CHUNK)], sem))
        for cp in copies:
            cp.wait()
        return carry

    lax.fori_loop(0, TC_ROWS * CPR // GROUP, gather_group_tc, 0)

    copies = []
    for k in range(SC_ROWS * CPR):
        p = TC_ROWS + k // CPR
        o = (k % CPR) * CHUNK
        copies.append(pltpu.async_copy(
            stag_sc_ref.at[idx_v.at[p, pl.ds(o, CHUNK)]],
            vals_v.at[p, pl.ds(o, CHUNK)], sem))
    for cp in copies:
        cp.wait()

    pltpu.sync_copy(vals_v, out_ref.at[:, pl.ds(tok0, TOK_W)])


def kernel(bits, tables):
    tc_relayout = pl.pallas_call(
        _tc_relayout_body,
        in_specs=[pl.BlockSpec(memory_space=pltpu.HBM)],
        out_specs=pl.BlockSpec(memory_space=pltpu.HBM),
        out_shape=jax.ShapeDtypeStruct((TC_ROWS * TABLE_SIZE,), jnp.float32),
    )
    mesh = plsc.VectorSubcoreMesh(core_axis_name="c", subcore_axis_name="s")
    sc_relayout = pl.kernel(
        _sc_relayout_body,
        mesh=mesh,
        out_type=jax.ShapeDtypeStruct((SC_ROWS * TABLE_SIZE,), jnp.float32),
        scratch_types=[
            pltpu.VMEM((SC_ROWS, CW), jnp.float32),
            pltpu.VMEM((SC_ROWS, CW), jnp.float32),
            pltpu.SemaphoreType.DMA,
            pltpu.SemaphoreType.DMA,
        ],
    )
    gather = pl.kernel(
        _gather_body,
        mesh=mesh,
        out_type=jax.ShapeDtypeStruct((N_BITS, BATCH), jnp.float32),
        scratch_types=[
            pltpu.VMEM((N_BITS, TOK_W), jnp.int32),
            pltpu.VMEM((N_BITS, TOK_W), jnp.int32),
            pltpu.VMEM((N_BITS, TOK_W), jnp.float32),
            pltpu.SemaphoreType.DMA,
        ],
    )
    stag_sc = sc_relayout(tables)
    stag_tc = tc_relayout(tables)
    out_t = gather(stag_tc, stag_sc, bits.T)
    return out_t.T


# speedup vs baseline: 18.7335x; 18.7335x over previous
import jax
import jax.numpy as jnp
from jax import lax
from jax.experimental import pallas as pl
from jax.experimental.pallas import tpu as pltpu
from jax.experimental.pallas import tpu_sc as plsc

N_BITS = 20
BATCH = 16384
TABLE_SIZE = 1 << N_BITS
FLAT = N_BITS * TABLE_SIZE

NC = 2
NS = 16
NW = NC * NS

CW = 2048
NCK = TABLE_SIZE // CW
CK_W = NCK // NW

TOK_W = BATCH // NW
CHUNK = 128
GROUP = 40
NGROUP = TOK_W * N_BITS // (CHUNK * GROUP)
CPR = TOK_W // CHUNK


def _relayout_body(table_ref, staging_ref, buf0, buf1, rsem, wsem):
    wid = lax.axis_index("s") * NC + lax.axis_index("c")
    ck0 = wid * CK_W
    bufs = (buf0, buf1)

    pltpu.async_copy(table_ref.at[:, pl.ds(ck0 * CW, CW)], buf0, rsem)
    pltpu.async_copy(table_ref.at[:, pl.ds((ck0 + 1) * CW, CW)], buf1, rsem)

    def step(k2, carry):
        for b in range(2):
            k = k2 * 2 + b
            cur = bufs[b]
            col = pl.multiple_of((ck0 + k) * CW, CW)
            pltpu.make_async_copy(
                table_ref.at[:, pl.ds(col, CW)], cur, rsem).wait()
            writes = []
            for p in range(N_BITS):
                writes.append(pltpu.async_copy(
                    cur.at[p],
                    staging_ref.at[
                        pl.ds(p * TABLE_SIZE + col, CW)],
                    wsem))
            for w in writes:
                w.wait()
            ncol = pl.multiple_of(
                (ck0 + lax.rem(k + 2, CK_W)) * CW, CW)
            pltpu.async_copy(table_ref.at[:, pl.ds(ncol, CW)], cur, rsem)
        return carry

    lax.fori_loop(0, CK_W // 2, step, 0)
    pltpu.make_async_copy(
        table_ref.at[:, pl.ds(ck0 * CW, CW)], buf0, rsem).wait()
    pltpu.make_async_copy(
        table_ref.at[:, pl.ds(ck0 * CW, CW)], buf1, rsem).wait()


def _gather_body(staging_ref, bitst_ref, out_ref, bits_v, idx_v, vals_v, sem):
    wid = lax.axis_index("s") * NC + lax.axis_index("c")
    tok0 = wid * TOK_W
    pltpu.sync_copy(bitst_ref.at[:, pl.ds(tok0, TOK_W)], bits_v)

    def compute_group(g, carry):
        o = pl.multiple_of(g * 16, 16)
        addr = jnp.zeros((16,), jnp.int32)
        for i in range(N_BITS):
            addr = addr + bits_v[i, pl.ds(o, 16)] * (1 << i)
        for p in range(N_BITS):
            idx_v[p, pl.ds(o, 16)] = addr + (p << N_BITS)
        return carry

    lax.fori_loop(0, TOK_W // 16, compute_group, 0)

    def gather_group(t, carry):
        copies = []
        for u in range(GROUP):
            k = t * GROUP + u
            p = lax.div(k, CPR)
            c = lax.rem(k, CPR)
            o = pl.multiple_of(c * CHUNK, CHUNK)
            copies.append(pltpu.async_copy(
                staging_ref.at[idx_v.at[p, pl.ds(o, CHUNK)]],
                vals_v.at[p, pl.ds(o, CHUNK)], sem))
        for cp in copies:
            cp.wait()
        return carry

    lax.fori_loop(0, NGROUP, gather_group, 0)

    pltpu.sync_copy(vals_v, out_ref.at[:, pl.ds(tok0, TOK_W)])


def kernel(bits, tables):
    mesh = plsc.VectorSubcoreMesh(core_axis_name="c", subcore_axis_name="s")
    relayout = pl.kernel(
        _relayout_body,
        mesh=mesh,
        out_type=jax.ShapeDtypeStruct((FLAT,), jnp.float32),
        scratch_types=[
            pltpu.VMEM((N_BITS, CW), jnp.float32),
            pltpu.VMEM((N_BITS, CW), jnp.float32),
            pltpu.SemaphoreType.DMA,
            pltpu.SemaphoreType.DMA,
        ],
    )
    gather = pl.kernel(
        _gather_body,
        mesh=mesh,
        out_type=jax.ShapeDtypeStruct((N_BITS, BATCH), jnp.float32),
        scratch_types=[
            pltpu.VMEM((N_BITS, TOK_W), jnp.int32),
            pltpu.VMEM((N_BITS, TOK_W), jnp.int32),
            pltpu.VMEM((N_BITS, TOK_W), jnp.float32),
            pltpu.SemaphoreType.DMA,
        ],
    )
    staging = relayout(tables)
    out_t = gather(staging, bits.T)
    return out_t.T


# speedup vs baseline: 18.9677x; 1.0125x over previous
import jax
import jax.numpy as jnp
from jax import lax
from jax.experimental import pallas as pl
from jax.experimental.pallas import tpu as pltpu
from jax.experimental.pallas import tpu_sc as plsc

N_BITS = 20
BATCH = 16384
TABLE_SIZE = 1 << N_BITS
FLAT = N_BITS * TABLE_SIZE

NC = 2
NS = 16
NW = NC * NS

CW = 2048
NCK = TABLE_SIZE // CW
CK_W = NCK // NW

TOK_W = BATCH // NW
CHUNK = 128
GROUP = 40
NGROUP = TOK_W * N_BITS // (CHUNK * GROUP)
CPR = TOK_W // CHUNK


def _relayout_body(table_ref, staging_ref, buf0, buf1, rsem, wsem):
    wid = lax.axis_index("s") * NC + lax.axis_index("c")
    ck0 = wid * CK_W
    bufs = (buf0, buf1)

    pltpu.async_copy(table_ref.at[:, pl.ds(ck0 * CW, CW)], buf0, rsem)
    pltpu.async_copy(table_ref.at[:, pl.ds((ck0 + 1) * CW, CW)], buf1, rsem)

    def step(k2, carry):
        for b in range(2):
            k = k2 * 2 + b
            cur = bufs[b]
            col = pl.multiple_of((ck0 + k) * CW, CW)
            pltpu.make_async_copy(
                table_ref.at[:, pl.ds(col, CW)], cur, rsem).wait()
            writes = []
            for p in range(N_BITS):
                writes.append(pltpu.async_copy(
                    cur.at[p],
                    staging_ref.at[
                        pl.ds(p * TABLE_SIZE + col, CW)],
                    wsem))
            for w in writes:
                w.wait()
            ncol = pl.multiple_of(
                (ck0 + lax.rem(k + 2, CK_W)) * CW, CW)
            pltpu.async_copy(table_ref.at[:, pl.ds(ncol, CW)], cur, rsem)
        return carry

    lax.fori_loop(0, CK_W // 2, step, 0)
    pltpu.make_async_copy(
        table_ref.at[:, pl.ds(ck0 * CW, CW)], buf0, rsem).wait()
    pltpu.make_async_copy(
        table_ref.at[:, pl.ds(ck0 * CW, CW)], buf1, rsem).wait()


def _tc_idx_body(bitst_ref, idx_ref):
    addr = jnp.zeros((1, BATCH), jnp.int32)
    for i in range(N_BITS):
        addr = addr + bitst_ref[pl.ds(i, 1), :] * (1 << i)
    for p in range(N_BITS):
        idx_ref[pl.ds(p, 1), :] = addr + (p << N_BITS)


def _gather_body(staging_ref, idxt_ref, out_ref, idx_v, vals_v, sem):
    wid = lax.axis_index("s") * NC + lax.axis_index("c")
    tok0 = wid * TOK_W
    pltpu.sync_copy(idxt_ref.at[:, pl.ds(tok0, TOK_W)], idx_v)

    def gather_group(t, carry):
        copies = []
        for u in range(GROUP):
            k = t * GROUP + u
            p = lax.div(k, CPR)
            c = lax.rem(k, CPR)
            o = pl.multiple_of(c * CHUNK, CHUNK)
            copies.append(pltpu.async_copy(
                staging_ref.at[idx_v.at[p, pl.ds(o, CHUNK)]],
                vals_v.at[p, pl.ds(o, CHUNK)], sem))
        for cp in copies:
            cp.wait()
        return carry

    lax.fori_loop(0, NGROUP, gather_group, 0)

    pltpu.sync_copy(vals_v, out_ref.at[:, pl.ds(tok0, TOK_W)])


def kernel(bits, tables):
    mesh = plsc.VectorSubcoreMesh(core_axis_name="c", subcore_axis_name="s")
    relayout = pl.kernel(
        _relayout_body,
        mesh=mesh,
        out_type=jax.ShapeDtypeStruct((FLAT,), jnp.float32),
        scratch_types=[
            pltpu.VMEM((N_BITS, CW), jnp.float32),
            pltpu.VMEM((N_BITS, CW), jnp.float32),
            pltpu.SemaphoreType.DMA,
            pltpu.SemaphoreType.DMA,
        ],
    )
    gather = pl.kernel(
        _gather_body,
        mesh=mesh,
        out_type=jax.ShapeDtypeStruct((N_BITS, BATCH), jnp.float32),
        scratch_types=[
            pltpu.VMEM((N_BITS, TOK_W), jnp.int32),
            pltpu.VMEM((N_BITS, TOK_W), jnp.float32),
            pltpu.SemaphoreType.DMA,
        ],
    )
    tc_idx = pl.pallas_call(
        _tc_idx_body,
        out_shape=jax.ShapeDtypeStruct((N_BITS, BATCH), jnp.int32),
    )
    staging = relayout(tables)
    idxt = tc_idx(bits.T)
    out_t = gather(staging, idxt)
    return out_t.T
